# Initial kernel scaffold; baseline (speedup 1.0000x reference)
import jax
import jax.numpy as jnp
from jax.experimental import pallas as pl


def kernel(concept_ids, relations, head_ids, tail_ids, concept_table, rel_table, W_s, W_n, W_r):
    raise NotImplementedError("write your pallas kernel here")


# speedup vs baseline: 6076.7039x; 6076.7039x over previous
import functools

import jax
import jax.numpy as jnp
from jax import lax
from jax.experimental import pallas as pl
from jax.experimental.pallas import tpu as pltpu
from jax.experimental.pallas import tpu_sc as plsc

NC = 2
NS = 16
NW = NC * NS


def _mesh():
    return plsc.VectorSubcoreMesh(core_axis_name="c", subcore_axis_name="s")


def _sc_gather_rows(table, idx_flat, ch=128):
    n = idx_flat.shape[0]
    e = table.shape[1]
    per_w = n // NW
    n_ch = per_w // ch

    @functools.partial(
        pl.kernel,
        out_type=jax.ShapeDtypeStruct((n, e), jnp.float32),
        mesh=_mesh(),
        scratch_types=[
            pltpu.VMEM((ch,), jnp.int32),
            pltpu.VMEM((ch, e), jnp.float32),
            pltpu.SemaphoreType.DMA,
        ],
    )
    def k(table_hbm, idx_hbm, out_hbm, idx_v, rows_v, sem):
        wid = lax.axis_index("s") * NC + lax.axis_index("c")
        base = wid * per_w

        def chunk(i, carry):
            off = base + i * ch
            pltpu.sync_copy(idx_hbm.at[pl.ds(off, ch)], idx_v)
            pltpu.async_copy(table_hbm.at[idx_v], rows_v, sem).wait()
            pltpu.sync_copy(rows_v, out_hbm.at[pl.ds(off, ch)])
            return carry

        lax.fori_loop(0, n_ch, chunk, 0)

    return k(table, idx_flat)


def _tc_node_linear(repr3, src_ids, dst_ids, rel_table, ws, wn, wr):
    bsz, m, e = repr3.shape
    p = src_ids.shape[1]
    r = rel_table.shape[0]
    src3 = src_ids.reshape(bsz, 1, p)
    dst3 = dst_ids.reshape(bsz, 1, p)

    def body(repr_ref, src_ref, dst_ref, rel_ref, ws_ref, wn_ref, wr_ref,
             node_ref, rel2_ref):
        x = repr_ref[0]
        srcv = src_ref[0]
        dstv = dst_ref[0]
        iota_m = lax.broadcasted_iota(jnp.int32, (m, p), 0)
        a_dst = (iota_m == dstv).astype(jnp.bfloat16)
        a_src = (iota_m == srcv).astype(jnp.bfloat16)
        dn_min = (((1,), (1,)), ((), ()))
        badj = lax.dot_general(a_dst, a_src, dn_min,
                               preferred_element_type=jnp.float32)
        deg = jnp.sum(a_dst.astype(jnp.float32), axis=1, keepdims=True)
        upd = lax.dot_general(badj, x, (((1,), (0,)), ((), ())),
                              preferred_element_type=jnp.float32)
        inv = 1.0 / jnp.maximum(deg, 1.0)
        acc = lax.dot_general(x, ws_ref[...], dn_min,
                              preferred_element_type=jnp.float32)
        acc += lax.dot_general(upd * inv, wn_ref[...], dn_min,
                               preferred_element_type=jnp.float32)
        node_ref[0] = jnp.maximum(acc, 0.0)

        @pl.when(pl.program_id(0) == 0)
        def _():
            rel2_ref[...] = lax.dot_general(rel_ref[...], wr_ref[...], dn_min,
                                            preferred_element_type=jnp.float32)

    return pl.pallas_call(
        body,
        grid=(bsz,),
        in_specs=[
            pl.BlockSpec((1, m, e), lambda b: (b, 0, 0)),
            pl.BlockSpec((1, 1, p), lambda b: (b, 0, 0)),
            pl.BlockSpec((1, 1, p), lambda b: (b, 0, 0)),
            pl.BlockSpec((r, e), lambda b: (0, 0)),
            pl.BlockSpec((e, e), lambda b: (0, 0)),
            pl.BlockSpec((e, e), lambda b: (0, 0)),
            pl.BlockSpec((e, e), lambda b: (0, 0)),
        ],
        out_specs=[
            pl.BlockSpec((1, m, e), lambda b: (b, 0, 0)),
            pl.BlockSpec((r, e), lambda b: (0, 0)),
        ],
        out_shape=[
            jax.ShapeDtypeStruct((bsz, m, e), jnp.float32),
            jax.ShapeDtypeStruct((r, e), jnp.float32),
        ],
    )(repr3, src3, dst3, rel_table, ws, wn, wr)


def _sc_assemble(node_flat, rel2, head_flat, tail_flat, rel_flat, m, mt, ch=128):
    n = head_flat.shape[0]
    e = node_flat.shape[1]
    per_w = n // NW
    n_ch = per_w // ch
    slots_per_b = mt

    @functools.partial(
        pl.kernel,
        out_type=jax.ShapeDtypeStruct((n, 3 * e), jnp.float32),
        mesh=_mesh(),
        scratch_types=[
            pltpu.VMEM((ch,), jnp.int32),
            pltpu.VMEM((ch,), jnp.int32),
            pltpu.VMEM((ch,), jnp.int32),
            pltpu.VMEM((ch,), jnp.int32),
            pltpu.VMEM((ch,), jnp.int32),
            pltpu.VMEM((ch, e), jnp.float32),
            pltpu.VMEM((ch, e), jnp.float32),
            pltpu.VMEM((ch, e), jnp.float32),
            pltpu.SemaphoreType.DMA,
            pltpu.SemaphoreType.DMA,
            pltpu.SemaphoreType.DMA,
        ],
    )
    def k(node_hbm, rel2_hbm, head_hbm, tail_hbm, rel_hbm, out_hbm,
          hid_v, tid_v, rid_v, gh_v, gt_v, bufh, buft, bufr,
          sem1, sem2, sem3):
        wid = lax.axis_index("s") * NC + lax.axis_index("c")
        base = wid * per_w

        def chunk(i, carry):
            off = base + i * ch
            b = lax.div(off, slots_per_b)
            pltpu.sync_copy(head_hbm.at[pl.ds(off, ch)], hid_v)
            pltpu.sync_copy(tail_hbm.at[pl.ds(off, ch)], tid_v)
            pltpu.sync_copy(rel_hbm.at[pl.ds(off, ch)], rid_v)
            boff = b * m
            for kk in range(ch // 16):
                sl = pl.ds(kk * 16, 16)
                gh_v[sl] = hid_v[sl] + boff
                gt_v[sl] = tid_v[sl] + boff
            d1 = pltpu.async_copy(node_hbm.at[gh_v], bufh, sem1)
            d2 = pltpu.async_copy(node_hbm.at[gt_v], buft, sem2)
            d3 = pltpu.async_copy(rel2_hbm.at[rid_v], bufr, sem3)
            d1.wait()
            d2.wait()
            d3.wait()
            pltpu.sync_copy(bufh, out_hbm.at[pl.ds(off, ch), pl.ds(0, e)])
            pltpu.sync_copy(bufr, out_hbm.at[pl.ds(off, ch), pl.ds(e, e)])
            pltpu.sync_copy(buft, out_hbm.at[pl.ds(off, ch), pl.ds(2 * e, e)])
            return carry

        lax.fori_loop(0, n_ch, chunk, 0)

    return k(node_flat, rel2, head_flat, tail_flat, rel_flat)


def kernel(concept_ids, relations, head_ids, tail_ids, concept_table,
           rel_table, W_s, W_n, W_r):
    bsz, m = concept_ids.shape
    mt = head_ids.shape[1]
    e = concept_table.shape[1]

    cids = concept_ids.astype(jnp.int32)
    rels = relations.astype(jnp.int32)
    hids = head_ids.astype(jnp.int32)
    tids = tail_ids.astype(jnp.int32)
    ws = W_s[-1].astype(jnp.float32)
    wn = W_n[-1].astype(jnp.float32)
    wr = W_r[-1].astype(jnp.float32)

    repr_flat = _sc_gather_rows(concept_table, cids.reshape(-1))

    src_ids = jnp.concatenate([hids, tids], axis=1)
    dst_ids = jnp.concatenate([tids, hids], axis=1)
    repr3 = repr_flat.reshape(bsz, m, e)
    node3, rel2 = _tc_node_linear(repr3, src_ids, dst_ids,
                                  rel_table.astype(jnp.float32), ws, wn, wr)

    out = _sc_assemble(node3.reshape(bsz * m, e), rel2,
                       hids.reshape(-1), tids.reshape(-1), rels.reshape(-1),
                       m, mt)
    return out.reshape(bsz, mt, 3 * e)


# speedup vs baseline: 6250.7808x; 1.0286x over previous
import functools

import jax
import jax.numpy as jnp
from jax import lax
from jax.experimental import pallas as pl
from jax.experimental.pallas import tpu as pltpu
from jax.experimental.pallas import tpu_sc as plsc

NC = 2
NS = 16
NW = NC * NS


def _mesh():
    return plsc.VectorSubcoreMesh(core_axis_name="c", subcore_axis_name="s")


def _sc_gather_rows(table, idx_flat, ch=128):
    n = idx_flat.shape[0]
    e = table.shape[1]
    per_w = n // NW
    n_ch = per_w // ch

    @functools.partial(
        pl.kernel,
        out_type=jax.ShapeDtypeStruct((n, e), jnp.float32),
        mesh=_mesh(),
        scratch_types=[
            pltpu.VMEM((ch,), jnp.int32),
            pltpu.VMEM((ch, e), jnp.float32),
            pltpu.SemaphoreType.DMA,
        ],
    )
    def k(table_hbm, idx_hbm, out_hbm, idx_v, rows_v, sem):
        wid = lax.axis_index("s") * NC + lax.axis_index("c")
        base = wid * per_w

        def chunk(i, carry):
            off = base + i * ch
            pltpu.sync_copy(idx_hbm.at[pl.ds(off, ch)], idx_v)
            pltpu.async_copy(table_hbm.at[idx_v], rows_v, sem).wait()
            pltpu.sync_copy(rows_v, out_hbm.at[pl.ds(off, ch)])
            return carry

        lax.fori_loop(0, n_ch, chunk, 0)

    return k(table, idx_flat)


def _tc_node_linear(repr3, src_ids, dst_ids, rel_table, ws, wn, wr):
    bsz, m, e = repr3.shape
    p = src_ids.shape[1]
    r = rel_table.shape[0]
    src3 = src_ids.reshape(bsz, 1, p)
    dst3 = dst_ids.reshape(bsz, 1, p)

    def body(repr_ref, src_ref, dst_ref, rel_ref, ws_ref, wn_ref, wr_ref,
             node_ref, rel2_ref):
        x = repr_ref[0]
        srcv = src_ref[0]
        dstv = dst_ref[0]
        iota_m = lax.broadcasted_iota(jnp.int32, (m, p), 0)
        a_dst = (iota_m == dstv).astype(jnp.bfloat16)
        a_src = (iota_m == srcv).astype(jnp.bfloat16)
        dn_min = (((1,), (1,)), ((), ()))
        badj = lax.dot_general(a_dst, a_src, dn_min,
                               preferred_element_type=jnp.float32)
        deg = jnp.sum(a_dst.astype(jnp.float32), axis=1, keepdims=True)
        upd = lax.dot_general(badj, x, (((1,), (0,)), ((), ())),
                              preferred_element_type=jnp.float32)
        inv = 1.0 / jnp.maximum(deg, 1.0)
        acc = lax.dot_general(x, ws_ref[...], dn_min,
                              preferred_element_type=jnp.float32)
        acc += lax.dot_general(upd * inv, wn_ref[...], dn_min,
                               preferred_element_type=jnp.float32)
        node_ref[0] = jnp.maximum(acc, 0.0)

        @pl.when(pl.program_id(0) == 0)
        def _():
            rel2_ref[...] = lax.dot_general(rel_ref[...], wr_ref[...], dn_min,
                                            preferred_element_type=jnp.float32)

    return pl.pallas_call(
        body,
        grid=(bsz,),
        in_specs=[
            pl.BlockSpec((1, m, e), lambda b: (b, 0, 0)),
            pl.BlockSpec((1, 1, p), lambda b: (b, 0, 0)),
            pl.BlockSpec((1, 1, p), lambda b: (b, 0, 0)),
            pl.BlockSpec((r, e), lambda b: (0, 0)),
            pl.BlockSpec((e, e), lambda b: (0, 0)),
            pl.BlockSpec((e, e), lambda b: (0, 0)),
            pl.BlockSpec((e, e), lambda b: (0, 0)),
        ],
        out_specs=[
            pl.BlockSpec((1, m, e), lambda b: (b, 0, 0)),
            pl.BlockSpec((r, e), lambda b: (0, 0)),
        ],
        out_shape=[
            jax.ShapeDtypeStruct((bsz, m, e), jnp.float32),
            jax.ShapeDtypeStruct((r, e), jnp.float32),
        ],
    )(repr3, src3, dst3, rel_table, ws, wn, wr)


def _sc_assemble(node_flat, rel2, ids_all, m, mt, ch=128):
    n3 = ids_all.shape[0]
    n = n3 // 3
    e = node_flat.shape[1]
    per_w = n3 // NW
    n_ch = per_w // ch
    kch = n // ch

    @functools.partial(
        pl.kernel,
        out_type=jax.ShapeDtypeStruct((n, 3 * e), jnp.float32),
        mesh=_mesh(),
        scratch_types=[
            pltpu.VMEM((per_w,), jnp.int32),
            pltpu.VMEM((ch,), jnp.int32),
            pltpu.VMEM((ch,), jnp.int32),
            pltpu.VMEM((ch, e), jnp.float32),
            pltpu.VMEM((ch, e), jnp.float32),
            pltpu.SemaphoreType.DMA,
            pltpu.SemaphoreType.DMA,
            pltpu.SemaphoreType.DMA,
            pltpu.SemaphoreType.DMA,
        ],
    )
    def k(node_hbm, rel2_hbm, ids_hbm, out_hbm,
          ids_v, idx0, idx1, buf0, buf1,
          sg0, sg1, sw0, sw1):
        wid = lax.axis_index("s") * NC + lax.axis_index("c")
        c0 = wid * n_ch
        idxs = (idx0, idx1)
        bufs = (buf0, buf1)
        sgs = (sg0, sg1)
        sws = (sw0, sw1)
        pltpu.sync_copy(ids_hbm.at[pl.ds(wid * per_w, per_w)], ids_v)

        def start_gather(r, ci):
            gcid = c0 + ci
            kind = lax.div(gcid, kch)
            row0 = lax.rem(gcid, kch) * ch
            boff = jnp.where(kind == 1, 0, lax.div(row0, mt) * m)
            for kk in range(ch // 16):
                sl = pl.ds(kk * 16, 16)
                idxs[r][sl] = ids_v[pl.ds(ci * ch + kk * 16, 16)] + boff

            @pl.when(kind == 1)
            def _():
                pltpu.async_copy(rel2_hbm.at[idxs[r]], bufs[r], sgs[r])

            @pl.when(kind != 1)
            def _():
                pltpu.async_copy(node_hbm.at[idxs[r]], bufs[r], sgs[r])

        def finish_and_write(r, ci):
            gcid = c0 + ci
            kind = lax.div(gcid, kch)
            row0 = lax.rem(gcid, kch) * ch
            pltpu.make_async_copy(node_hbm.at[idxs[r]], bufs[r], sgs[r]).wait()
            for kk, col in enumerate((0, e, 2 * e)):
                @pl.when(kind == kk)
                def _():
                    pltpu.async_copy(
                        bufs[r], out_hbm.at[pl.ds(row0, ch), pl.ds(col, e)],
                        sws[r])

        def wait_write(r):
            pltpu.make_async_copy(
                bufs[r], out_hbm.at[pl.ds(0, ch), pl.ds(0, e)], sws[r]).wait()

        def body2(o, carry):
            for r in (0, 1):
                ci = 2 * o + r

                @pl.when(ci >= 2)
                def _():
                    wait_write(r)

                start_gather(r, ci)

                @pl.when(ci >= 1)
                def _():
                    finish_and_write(1 - r, ci - 1)

            return carry

        lax.fori_loop(0, n_ch // 2, body2, 0)
        finish_and_write((n_ch - 1) % 2, n_ch - 1)
        wait_write(0)
        wait_write(1)

    return k(node_flat, rel2, ids_all)


def kernel(concept_ids, relations, head_ids, tail_ids, concept_table,
           rel_table, W_s, W_n, W_r):
    bsz, m = concept_ids.shape
    mt = head_ids.shape[1]
    e = concept_table.shape[1]

    cids = concept_ids.astype(jnp.int32)
    rels = relations.astype(jnp.int32)
    hids = head_ids.astype(jnp.int32)
    tids = tail_ids.astype(jnp.int32)
    ws = W_s[-1].astype(jnp.float32)
    wn = W_n[-1].astype(jnp.float32)
    wr = W_r[-1].astype(jnp.float32)

    repr_flat = _sc_gather_rows(concept_table, cids.reshape(-1))

    src_ids = jnp.concatenate([hids, tids], axis=1)
    dst_ids = jnp.concatenate([tids, hids], axis=1)
    repr3 = repr_flat.reshape(bsz, m, e)
    node3, rel2 = _tc_node_linear(repr3, src_ids, dst_ids,
                                  rel_table.astype(jnp.float32), ws, wn, wr)

    ids_all = jnp.concatenate([hids.reshape(-1), rels.reshape(-1),
                               tids.reshape(-1)])
    out = _sc_assemble(node3.reshape(bsz * m, e), rel2, ids_all, m, mt)
    return out.reshape(bsz, mt, 3 * e)


# speedup vs baseline: 14647.7042x; 2.3433x over previous
import functools

import jax
import jax.numpy as jnp
from jax import lax
from jax.experimental import pallas as pl
from jax.experimental.pallas import tpu as pltpu
from jax.experimental.pallas import tpu_sc as plsc

NC = 2
NS = 16
NW = NC * NS


def _mesh():
    return plsc.VectorSubcoreMesh(core_axis_name="c", subcore_axis_name="s")


def _sc_gather_rows(table, idx_flat, ch=128):
    n = idx_flat.shape[0]
    e = table.shape[1]
    per_w = n // NW
    n_ch = per_w // ch

    @functools.partial(
        pl.kernel,
        out_type=jax.ShapeDtypeStruct((n, e), jnp.float32),
        mesh=_mesh(),
        scratch_types=[
            pltpu.VMEM((ch,), jnp.int32),
            pltpu.VMEM((ch, e), jnp.float32),
            pltpu.SemaphoreType.DMA,
        ],
    )
    def k(table_hbm, idx_hbm, out_hbm, idx_v, rows_v, sem):
        wid = lax.axis_index("s") * NC + lax.axis_index("c")
        base = wid * per_w

        def chunk(i, carry):
            off = base + i * ch
            pltpu.sync_copy(idx_hbm.at[pl.ds(off, ch)], idx_v)
            pltpu.async_copy(table_hbm.at[idx_v], rows_v, sem).wait()
            pltpu.sync_copy(rows_v, out_hbm.at[pl.ds(off, ch)])
            return carry

        lax.fori_loop(0, n_ch, chunk, 0)

    return k(table, idx_flat)


def _tc_node_linear(repr3, src_ids, dst_ids, head_ids, tail_ids, rel_ids,
                    rel_table, ws, wn, wr):
    bsz, m, e = repr3.shape
    p = src_ids.shape[1]
    mt = head_ids.shape[1]
    r = rel_table.shape[0]
    src3 = src_ids.reshape(bsz, 1, p)
    dst3 = dst_ids.reshape(bsz, 1, p)
    head3 = head_ids.reshape(bsz, 1, mt)
    tail3 = tail_ids.reshape(bsz, 1, mt)
    rel3 = rel_ids.reshape(bsz, 1, mt)
    f32 = jnp.float32
    bf16 = jnp.bfloat16

    def body(repr_ref, src_ref, dst_ref, head_ref, tail_ref, relid_ref,
             rel_ref, ws_ref, wn_ref, wr_ref, out_ref):
        x = repr_ref[0]
        srcv = src_ref[0]
        dstv = dst_ref[0]
        iota_m = lax.broadcasted_iota(jnp.int32, (m, p), 0)
        a_dst = (iota_m == dstv).astype(bf16)
        a_src = (iota_m == srcv).astype(bf16)
        dn_min = (((1,), (1,)), ((), ()))
        badj = lax.dot_general(a_dst, a_src, dn_min,
                               preferred_element_type=f32)
        deg = jnp.sum(badj, axis=1, keepdims=True)
        upd = lax.dot_general(badj.astype(bf16), x.astype(bf16),
                              (((1,), (0,)), ((), ())),
                              preferred_element_type=f32)
        inv = 1.0 / jnp.maximum(deg, 1.0)
        acc = lax.dot_general(x, ws_ref[...], dn_min,
                              preferred_element_type=f32)
        acc += lax.dot_general(upd * inv, wn_ref[...], dn_min,
                               preferred_element_type=f32)
        node = jnp.maximum(acc, 0.0).astype(bf16)

        rel2 = lax.dot_general(rel_ref[...], wr_ref[...], dn_min,
                               preferred_element_type=f32).astype(bf16)

        dn_sub = (((0,), (0,)), ((), ()))
        iota_g = lax.broadcasted_iota(jnp.int32, (m, mt), 0)
        ah = (iota_g == head_ref[0]).astype(bf16)
        at = (iota_g == tail_ref[0]).astype(bf16)
        iota_r = lax.broadcasted_iota(jnp.int32, (r, mt), 0)
        ar = (iota_r == relid_ref[0]).astype(bf16)
        out_h = lax.dot_general(ah, node, dn_sub, preferred_element_type=f32)
        out_r = lax.dot_general(ar, rel2, dn_sub, preferred_element_type=f32)
        out_t = lax.dot_general(at, node, dn_sub, preferred_element_type=f32)
        out_ref[0] = lax.concatenate([out_h, out_r, out_t], 1)

    return pl.pallas_call(
        body,
        grid=(bsz,),
        in_specs=[
            pl.BlockSpec((1, m, e), lambda b: (b, 0, 0)),
            pl.BlockSpec((1, 1, p), lambda b: (b, 0, 0)),
            pl.BlockSpec((1, 1, p), lambda b: (b, 0, 0)),
            pl.BlockSpec((1, 1, mt), lambda b: (b, 0, 0)),
            pl.BlockSpec((1, 1, mt), lambda b: (b, 0, 0)),
            pl.BlockSpec((1, 1, mt), lambda b: (b, 0, 0)),
            pl.BlockSpec((r, e), lambda b: (0, 0)),
            pl.BlockSpec((e, e), lambda b: (0, 0)),
            pl.BlockSpec((e, e), lambda b: (0, 0)),
            pl.BlockSpec((e, e), lambda b: (0, 0)),
        ],
        out_specs=pl.BlockSpec((1, mt, 3 * e), lambda b: (b, 0, 0)),
        out_shape=jax.ShapeDtypeStruct((bsz, mt, 3 * e), f32),
    )(repr3, src3, dst3, head3, tail3, rel3, rel_table, ws, wn, wr)


def _sc_assemble(node_flat, rel2, ids_all, m, mt, ch=128):
    n3 = ids_all.shape[0]
    n = n3 // 3
    e = node_flat.shape[1]
    per_w = n3 // NW
    n_ch = per_w // ch
    kch = n // ch

    @functools.partial(
        pl.kernel,
        out_type=jax.ShapeDtypeStruct((n, 3 * e), jnp.float32),
        mesh=_mesh(),
        scratch_types=[
            pltpu.VMEM((per_w,), jnp.int32),
            pltpu.VMEM((ch,), jnp.int32),
            pltpu.VMEM((ch,), jnp.int32),
            pltpu.VMEM((ch, e), jnp.float32),
            pltpu.VMEM((ch, e), jnp.float32),
            pltpu.SemaphoreType.DMA,
            pltpu.SemaphoreType.DMA,
            pltpu.SemaphoreType.DMA,
            pltpu.SemaphoreType.DMA,
        ],
    )
    def k(node_hbm, rel2_hbm, ids_hbm, out_hbm,
          ids_v, idx0, idx1, buf0, buf1,
          sg0, sg1, sw0, sw1):
        wid = lax.axis_index("s") * NC + lax.axis_index("c")
        c0 = wid * n_ch
        idxs = (idx0, idx1)
        bufs = (buf0, buf1)
        sgs = (sg0, sg1)
        sws = (sw0, sw1)
        pltpu.sync_copy(ids_hbm.at[pl.ds(wid * per_w, per_w)], ids_v)

        def start_gather(r, ci):
            gcid = c0 + ci
            kind = lax.div(gcid, kch)
            row0 = lax.rem(gcid, kch) * ch
            boff = jnp.where(kind == 1, 0, lax.div(row0, mt) * m)
            for kk in range(ch // 16):
                sl = pl.ds(kk * 16, 16)
                idxs[r][sl] = ids_v[pl.ds(ci * ch + kk * 16, 16)] + boff

            @pl.when(kind == 1)
            def _():
                pltpu.async_copy(rel2_hbm.at[idxs[r]], bufs[r], sgs[r])

            @pl.when(kind != 1)
            def _():
                pltpu.async_copy(node_hbm.at[idxs[r]], bufs[r], sgs[r])

        def finish_and_write(r, ci):
            gcid = c0 + ci
            kind = lax.div(gcid, kch)
            row0 = lax.rem(gcid, kch) * ch
            pltpu.make_async_copy(node_hbm.at[idxs[r]], bufs[r], sgs[r]).wait()
            for kk, col in enumerate((0, e, 2 * e)):
                @pl.when(kind == kk)
                def _():
                    pltpu.async_copy(
                        bufs[r], out_hbm.at[pl.ds(row0, ch), pl.ds(col, e)],
                        sws[r])

        def wait_write(r):
            pltpu.make_async_copy(
                bufs[r], out_hbm.at[pl.ds(0, ch), pl.ds(0, e)], sws[r]).wait()

        def body2(o, carry):
            for r in (0, 1):
                ci = 2 * o + r

                @pl.when(ci >= 2)
                def _():
                    wait_write(r)

                start_gather(r, ci)

                @pl.when(ci >= 1)
                def _():
                    finish_and_write(1 - r, ci - 1)

            return carry

        lax.fori_loop(0, n_ch // 2, body2, 0)
        finish_and_write((n_ch - 1) % 2, n_ch - 1)
        wait_write(0)
        wait_write(1)

    return k(node_flat, rel2, ids_all)


def kernel(concept_ids, relations, head_ids, tail_ids, concept_table,
           rel_table, W_s, W_n, W_r):
    bsz, m = concept_ids.shape
    mt = head_ids.shape[1]
    e = concept_table.shape[1]

    cids = concept_ids.astype(jnp.int32)
    rels = relations.astype(jnp.int32)
    hids = head_ids.astype(jnp.int32)
    tids = tail_ids.astype(jnp.int32)
    ws = W_s[-1].astype(jnp.float32)
    wn = W_n[-1].astype(jnp.float32)
    wr = W_r[-1].astype(jnp.float32)

    repr_flat = _sc_gather_rows(concept_table, cids.reshape(-1))

    src_ids = jnp.concatenate([hids, tids], axis=1)
    dst_ids = jnp.concatenate([tids, hids], axis=1)
    repr3 = repr_flat.reshape(bsz, m, e)
    return _tc_node_linear(repr3, src_ids, dst_ids, hids, tids, rels,
                           rel_table.astype(jnp.float32), ws, wn, wr)


# speedup vs baseline: 14830.1466x; 1.0125x over previous
import functools

import jax
import jax.numpy as jnp
from jax import lax
from jax.experimental import pallas as pl
from jax.experimental.pallas import tpu as pltpu
from jax.experimental.pallas import tpu_sc as plsc

NC = 2
NS = 16
NW = NC * NS


def _mesh():
    return plsc.VectorSubcoreMesh(core_axis_name="c", subcore_axis_name="s")


def _sc_gather_rows(table, idx_flat, ch=128):
    n = idx_flat.shape[0]
    e = table.shape[1]
    per_w = n // NW
    n_ch = per_w // ch

    @functools.partial(
        pl.kernel,
        out_type=jax.ShapeDtypeStruct((n, e), jnp.float32),
        mesh=_mesh(),
        scratch_types=[
            pltpu.VMEM((ch,), jnp.int32),
            pltpu.VMEM((ch, e), jnp.float32),
            pltpu.SemaphoreType.DMA,
        ],
    )
    def k(table_hbm, idx_hbm, out_hbm, idx_v, rows_v, sem):
        wid = lax.axis_index("s") * NC + lax.axis_index("c")
        base = wid * per_w

        def chunk(i, carry):
            off = base + i * ch
            pltpu.sync_copy(idx_hbm.at[pl.ds(off, ch)], idx_v)
            pltpu.async_copy(table_hbm.at[idx_v], rows_v, sem).wait()
            pltpu.sync_copy(rows_v, out_hbm.at[pl.ds(off, ch)])
            return carry

        lax.fori_loop(0, n_ch, chunk, 0)

    return k(table, idx_flat)


def _tc_node_linear(repr3, head_ids, tail_ids, rel_ids,
                    rel_table, ws, wn, wr):
    bsz, m, e = repr3.shape
    mt = head_ids.shape[1]
    r = rel_table.shape[0]
    head3 = head_ids.reshape(bsz, 1, mt)
    tail3 = tail_ids.reshape(bsz, 1, mt)
    rel3 = rel_ids.reshape(bsz, 1, mt)
    f32 = jnp.float32
    bf16 = jnp.bfloat16

    def body(repr_ref, head_ref, tail_ref, relid_ref,
             rel_ref, ws_ref, wn_ref, wr_ref, out_ref):
        x16 = repr_ref[0].astype(bf16)
        iota_g = lax.broadcasted_iota(jnp.int32, (m, mt), 0)
        ah = (iota_g == head_ref[0]).astype(bf16)
        at = (iota_g == tail_ref[0]).astype(bf16)
        dn_min = (((1,), (1,)), ((), ()))
        badj = lax.dot_general(at, ah, dn_min, preferred_element_type=f32)
        badj += lax.dot_general(ah, at, dn_min, preferred_element_type=f32)
        deg = jnp.sum(badj, axis=1, keepdims=True)
        upd = lax.dot_general(badj.astype(bf16), x16,
                              (((1,), (0,)), ((), ())),
                              preferred_element_type=f32)
        inv = 1.0 / jnp.maximum(deg, 1.0)
        acc = lax.dot_general(x16, ws_ref[...], dn_min,
                              preferred_element_type=f32)
        acc += lax.dot_general((upd * inv).astype(bf16), wn_ref[...], dn_min,
                               preferred_element_type=f32)
        node = jnp.maximum(acc, 0.0).astype(bf16)

        rel2 = lax.dot_general(rel_ref[...], wr_ref[...], dn_min,
                               preferred_element_type=f32).astype(bf16)

        dn_sub = (((0,), (0,)), ((), ()))
        iota_r = lax.broadcasted_iota(jnp.int32, (r, mt), 0)
        ar = (iota_r == relid_ref[0]).astype(bf16)
        out_ref[0, :, 0:e] = lax.dot_general(ah, node, dn_sub,
                                             preferred_element_type=f32)
        out_ref[0, :, e:2 * e] = lax.dot_general(ar, rel2, dn_sub,
                                                 preferred_element_type=f32)
        out_ref[0, :, 2 * e:3 * e] = lax.dot_general(at, node, dn_sub,
                                                     preferred_element_type=f32)

    return pl.pallas_call(
        body,
        grid=(bsz,),
        in_specs=[
            pl.BlockSpec((1, m, e), lambda b: (b, 0, 0)),
            pl.BlockSpec((1, 1, mt), lambda b: (b, 0, 0)),
            pl.BlockSpec((1, 1, mt), lambda b: (b, 0, 0)),
            pl.BlockSpec((1, 1, mt), lambda b: (b, 0, 0)),
            pl.BlockSpec((r, e), lambda b: (0, 0)),
            pl.BlockSpec((e, e), lambda b: (0, 0)),
            pl.BlockSpec((e, e), lambda b: (0, 0)),
            pl.BlockSpec((e, e), lambda b: (0, 0)),
        ],
        out_specs=pl.BlockSpec((1, mt, 3 * e), lambda b: (b, 0, 0)),
        out_shape=jax.ShapeDtypeStruct((bsz, mt, 3 * e), f32),
    )(repr3, head3, tail3, rel3, rel_table, ws, wn, wr)


def kernel(concept_ids, relations, head_ids, tail_ids, concept_table,
           rel_table, W_s, W_n, W_r):
    bsz, m = concept_ids.shape
    mt = head_ids.shape[1]
    e = concept_table.shape[1]

    cids = concept_ids.astype(jnp.int32)
    rels = relations.astype(jnp.int32)
    hids = head_ids.astype(jnp.int32)
    tids = tail_ids.astype(jnp.int32)
    ws = W_s[-1].astype(jnp.float32)
    wn = W_n[-1].astype(jnp.float32)
    wr = W_r[-1].astype(jnp.float32)

    repr_flat = _sc_gather_rows(concept_table, cids.reshape(-1))

    repr3 = repr_flat.reshape(bsz, m, e)
    return _tc_node_linear(repr3, hids, tids, rels,
                           rel_table.astype(jnp.float32),
                           ws.astype(jnp.bfloat16), wn.astype(jnp.bfloat16),
                           wr)


# speedup vs baseline: 15254.0628x; 1.0286x over previous
import functools

import jax
import jax.numpy as jnp
from jax import lax
from jax.experimental import pallas as pl
from jax.experimental.pallas import tpu as pltpu
from jax.experimental.pallas import tpu_sc as plsc

NC = 2
NS = 16
NW = NC * NS


def _mesh():
    return plsc.VectorSubcoreMesh(core_axis_name="c", subcore_axis_name="s")


def _sc_gather_rows(table, idx_flat, ch=128):
    n = idx_flat.shape[0]
    e = table.shape[1]
    per_w = n // NW
    n_ch = per_w // ch

    @functools.partial(
        pl.kernel,
        out_type=jax.ShapeDtypeStruct((n, e), jnp.float32),
        mesh=_mesh(),
        scratch_types=[
            pltpu.VMEM((per_w,), jnp.int32),
            pltpu.VMEM((ch,), jnp.int32),
            pltpu.VMEM((ch,), jnp.int32),
            pltpu.VMEM((ch, e), jnp.float32),
            pltpu.VMEM((ch, e), jnp.float32),
            pltpu.SemaphoreType.DMA,
            pltpu.SemaphoreType.DMA,
            pltpu.SemaphoreType.DMA,
            pltpu.SemaphoreType.DMA,
        ],
    )
    def k(table_hbm, idx_hbm, out_hbm, ids_v, idx0, idx1, buf0, buf1,
          sg0, sg1, sw0, sw1):
        wid = lax.axis_index("s") * NC + lax.axis_index("c")
        base = wid * per_w
        idxs = (idx0, idx1)
        bufs = (buf0, buf1)
        sgs = (sg0, sg1)
        sws = (sw0, sw1)
        pltpu.sync_copy(idx_hbm.at[pl.ds(base, per_w)], ids_v)

        def start_gather(r, ci):
            for kk in range(ch // 16):
                sl = pl.ds(kk * 16, 16)
                idxs[r][sl] = ids_v[pl.ds(ci * ch + kk * 16, 16)]
            pltpu.async_copy(table_hbm.at[idxs[r]], bufs[r], sgs[r])

        def finish_and_write(r, ci):
            pltpu.make_async_copy(table_hbm.at[idxs[r]], bufs[r],
                                  sgs[r]).wait()
            pltpu.async_copy(bufs[r], out_hbm.at[pl.ds(base + ci * ch, ch)],
                             sws[r])

        def wait_write(r):
            pltpu.make_async_copy(bufs[r], out_hbm.at[pl.ds(0, ch)],
                                  sws[r]).wait()

        def body2(o, carry):
            for r in (0, 1):
                ci = 2 * o + r

                @pl.when(ci >= 2)
                def _():
                    wait_write(r)

                start_gather(r, ci)

                @pl.when(ci >= 1)
                def _():
                    finish_and_write(1 - r, ci - 1)

            return carry

        lax.fori_loop(0, n_ch // 2, body2, 0)
        finish_and_write((n_ch - 1) % 2, n_ch - 1)
        wait_write(0)
        wait_write(1)

    return k(table, idx_flat)


def _tc_node_linear(repr3, head_ids, tail_ids, rel_ids,
                    rel_table, ws, wn, wr):
    bsz, m, e = repr3.shape
    mt = head_ids.shape[1]
    r = rel_table.shape[0]
    head3 = head_ids.reshape(bsz, 1, mt)
    tail3 = tail_ids.reshape(bsz, 1, mt)
    rel3 = rel_ids.reshape(bsz, 1, mt)
    f32 = jnp.float32
    bf16 = jnp.bfloat16

    def body(repr_ref, head_ref, tail_ref, relid_ref,
             rel_ref, ws_ref, wn_ref, wr_ref, out_ref):
        x16 = repr_ref[0].astype(bf16)
        iota_g = lax.broadcasted_iota(jnp.int32, (m, mt), 0)
        ah = (iota_g == head_ref[0]).astype(bf16)
        at = (iota_g == tail_ref[0]).astype(bf16)
        dn_min = (((1,), (1,)), ((), ()))
        badj = lax.dot_general(at, ah, dn_min, preferred_element_type=f32)
        badj += lax.dot_general(ah, at, dn_min, preferred_element_type=f32)
        deg = jnp.sum(badj, axis=1, keepdims=True)
        upd = lax.dot_general(badj.astype(bf16), x16,
                              (((1,), (0,)), ((), ())),
                              preferred_element_type=f32)
        inv = 1.0 / jnp.maximum(deg, 1.0)
        acc = lax.dot_general(x16, ws_ref[...], dn_min,
                              preferred_element_type=f32)
        acc += lax.dot_general((upd * inv).astype(bf16), wn_ref[...], dn_min,
                               preferred_element_type=f32)
        node = jnp.maximum(acc, 0.0).astype(bf16)

        rel2 = lax.dot_general(rel_ref[...], wr_ref[...], dn_min,
                               preferred_element_type=f32).astype(bf16)

        dn_sub = (((0,), (0,)), ((), ()))
        iota_r = lax.broadcasted_iota(jnp.int32, (r, mt), 0)
        ar = (iota_r == relid_ref[0]).astype(bf16)
        out_ref[0, :, 0:e] = lax.dot_general(ah, node, dn_sub,
                                             preferred_element_type=f32)
        out_ref[0, :, e:2 * e] = lax.dot_general(ar, rel2, dn_sub,
                                                 preferred_element_type=f32)
        out_ref[0, :, 2 * e:3 * e] = lax.dot_general(at, node, dn_sub,
                                                     preferred_element_type=f32)

    return pl.pallas_call(
        body,
        grid=(bsz,),
        in_specs=[
            pl.BlockSpec((1, m, e), lambda b: (b, 0, 0)),
            pl.BlockSpec((1, 1, mt), lambda b: (b, 0, 0)),
            pl.BlockSpec((1, 1, mt), lambda b: (b, 0, 0)),
            pl.BlockSpec((1, 1, mt), lambda b: (b, 0, 0)),
            pl.BlockSpec((r, e), lambda b: (0, 0)),
            pl.BlockSpec((e, e), lambda b: (0, 0)),
            pl.BlockSpec((e, e), lambda b: (0, 0)),
            pl.BlockSpec((e, e), lambda b: (0, 0)),
        ],
        out_specs=pl.BlockSpec((1, mt, 3 * e), lambda b: (b, 0, 0)),
        out_shape=jax.ShapeDtypeStruct((bsz, mt, 3 * e), f32),
    )(repr3, head3, tail3, rel3, rel_table, ws, wn, wr)


def kernel(concept_ids, relations, head_ids, tail_ids, concept_table,
           rel_table, W_s, W_n, W_r):
    bsz, m = concept_ids.shape
    mt = head_ids.shape[1]
    e = concept_table.shape[1]

    cids = concept_ids.astype(jnp.int32)
    rels = relations.astype(jnp.int32)
    hids = head_ids.astype(jnp.int32)
    tids = tail_ids.astype(jnp.int32)
    ws = W_s[-1].astype(jnp.float32)
    wn = W_n[-1].astype(jnp.float32)
    wr = W_r[-1].astype(jnp.float32)

    repr_flat = _sc_gather_rows(concept_table, cids.reshape(-1))

    repr3 = repr_flat.reshape(bsz, m, e)
    return _tc_node_linear(repr3, hids, tids, rels,
                           rel_table.astype(jnp.float32),
                           ws.astype(jnp.bfloat16), wn.astype(jnp.bfloat16),
                           wr)
